# baseline (speedup 1.0000x reference)
import functools

import jax
import jax.numpy as jnp
from jax import lax
from jax.experimental import pallas as pl
from jax.experimental.pallas import tpu as pltpu
from jax.experimental.pallas import tpu_sc as plsc

_NC, _NS = 2, 16


def _make_sc_copy(seq_len, cols, dtype):
    nw = _NC * _NS
    rows_w = seq_len // nw
    chunk = rows_w // 2
    mesh = plsc.VectorSubcoreMesh(core_axis_name="c", subcore_axis_name="s")

    @functools.partial(
        pl.kernel,
        mesh=mesh,
        out_type=jax.ShapeDtypeStruct((seq_len, cols), dtype),
        scratch_types=[
            pltpu.VMEM((chunk, cols), dtype),
            pltpu.VMEM((chunk, cols), dtype),
            pltpu.SemaphoreType.DMA,
            pltpu.SemaphoreType.DMA,
            pltpu.SemaphoreType.DMA,
            pltpu.SemaphoreType.DMA,
        ],
    )
    def k(w_hbm, out_hbm, buf0, buf1, r0, r1, w0, w1):
        wid = lax.axis_index("s") * _NC + lax.axis_index("c")
        base = wid * rows_w
        rd0 = pltpu.make_async_copy(w_hbm.at[pl.ds(base, chunk), :], buf0, r0)
        rd1 = pltpu.make_async_copy(w_hbm.at[pl.ds(base + chunk, chunk), :], buf1, r1)
        rd0.start()
        rd1.start()
        rd0.wait()
        wr0 = pltpu.make_async_copy(buf0, out_hbm.at[pl.ds(base, chunk), :], w0)
        wr0.start()
        rd1.wait()
        wr1 = pltpu.make_async_copy(buf1, out_hbm.at[pl.ds(base + chunk, chunk), :], w1)
        wr1.start()
        wr0.wait()
        wr1.wait()

    return k


def kernel(indices, weight):
    seq_len = indices.shape[-2]
    cols = weight.shape[1]
    return _make_sc_copy(seq_len, cols, weight.dtype)(weight)


# speedup vs baseline: 4.2835x; 4.2835x over previous
import jax
import jax.numpy as jnp
from jax.experimental import pallas as pl
from jax.experimental.pallas import tpu as pltpu

_NCHUNK = 2


def _copy_body(seq_len, cols, nchunk):
    rows = seq_len // nchunk

    def body(w_hbm, o_hbm, vmem, rsem, wsem):
        reads = []
        for i in range(nchunk):
            sl = pl.ds(i * rows, rows)
            reads.append(pltpu.make_async_copy(w_hbm.at[sl, :], vmem.at[i], rsem.at[i]))
        for r in reads:
            r.start()
        writes = []
        for i in range(nchunk):
            sl = pl.ds(i * rows, rows)
            reads[i].wait()
            w = pltpu.make_async_copy(vmem.at[i], o_hbm.at[sl, :], wsem.at[i])
            w.start()
            writes.append(w)
        for w in writes:
            w.wait()

    return body


def kernel(indices, weight):
    seq_len = indices.shape[-2]
    cols = weight.shape[1]
    nchunk = _NCHUNK
    while seq_len % nchunk:
        nchunk //= 2
    rows = seq_len // nchunk
    return pl.pallas_call(
        _copy_body(seq_len, cols, nchunk),
        out_shape=jax.ShapeDtypeStruct((seq_len, cols), weight.dtype),
        in_specs=[pl.BlockSpec(memory_space=pl.ANY)],
        out_specs=pl.BlockSpec(memory_space=pl.ANY),
        scratch_shapes=[
            pltpu.VMEM((nchunk, rows, cols), weight.dtype),
            pltpu.SemaphoreType.DMA((nchunk,)),
            pltpu.SemaphoreType.DMA((nchunk,)),
        ],
    )(weight)
